# Initial kernel scaffold; baseline (speedup 1.0000x reference)
import jax
import jax.numpy as jnp
from jax.experimental import pallas as pl


def kernel(x, edge_index, batch, W_l0, b_l0, W_r0, W_l1, b_l1, W_r1, W_l2, b_l2, W_r2):
    raise NotImplementedError("write your pallas kernel here")


# speedup vs baseline: 1.4935x; 1.4935x over previous
import functools

import jax
import jax.numpy as jnp
from jax import lax
from jax.experimental import pallas as pl
from jax.experimental.pallas import tpu as pltpu
from jax.experimental.pallas import tpu_sc as plsc

N = 10000
E = 160000
D = 300
DP = 384
SW = 128
NS = 3
G = 64

LCH = 128
NT = 16
NW = 32
CPT = 40
EPAD = CPT * LCH * NW
TOT_CHUNKS = EPAD // LCH

NPAD = 10112
RPT = NPAD // NT
RPT_LAST = N - (NT - 1) * RPT
ZR = 79

CRPT = 640
CNPAD = CRPT * NT

_SC_PARAMS = pltpu.CompilerParams(use_tc_tiling_on_sc=False)
_MESH = plsc.VectorSubcoreMesh(core_axis_name="c", subcore_axis_name="s")


def _seg_sum_body(ha, hb, hc, src2, dst2, zrows, out0, out1,
                  scur, dcur, rows, zbuf, acc, sem):
  c = lax.axis_index("c")
  s = lax.axis_index("s")
  w = s * 2 + c
  base = w * CPT
  pltpu.sync_copy(zrows, zbuf)
  my0 = s * RPT

  for k, h_ref in enumerate((ha, hb, hc)):
    for i in range(RPT // ZR):
      pltpu.sync_copy(zbuf, acc.at[pl.ds(my0 + i * ZR, ZR)])
    plsc.subcore_barrier()

    def chunk(j, carry):
      pltpu.sync_copy(src2.at[base + j], scur)
      pltpu.sync_copy(dst2.at[base + j], dcur)
      pltpu.async_copy(h_ref.at[scur], rows, sem).wait()
      pltpu.sync_copy(rows, acc.at[dcur], add=True)
      return carry
    lax.fori_loop(0, CPT, chunk, 0, unroll=False)
    plsc.subcore_barrier()

    def cp(oref):
      @pl.when(s < NT - 1)
      def _():
        pltpu.sync_copy(acc.at[pl.ds(my0, RPT)],
                        oref.at[k, pl.ds(my0, RPT)])
      @pl.when(s == NT - 1)
      def _():
        pltpu.sync_copy(acc.at[pl.ds((NT - 1) * RPT, RPT_LAST)],
                        oref.at[k, pl.ds((NT - 1) * RPT, RPT_LAST)])

    @pl.when(c == 0)
    def _():
      cp(out0)

    @pl.when(c == 1)
    def _():
      cp(out1)


_seg_sum = pl.kernel(
    _seg_sum_body,
    out_type=[jax.ShapeDtypeStruct((NS, N, SW), jnp.float32),
              jax.ShapeDtypeStruct((NS, N, SW), jnp.float32)],
    mesh=_MESH,
    scratch_types=[
        pltpu.VMEM((LCH,), jnp.int32),
        pltpu.VMEM((LCH,), jnp.int32),
        pltpu.VMEM((LCH, SW), jnp.float32),
        pltpu.VMEM((ZR, SW), jnp.float32),
        pltpu.VMEM_SHARED((NPAD, SW), jnp.float32),
        pltpu.SemaphoreType.DMA,
    ],
    compiler_params=_SC_PARAMS,
)


def _count_body(dst2, out, dcur, zbuf, ones_v, acc):
  c = lax.axis_index("c")
  s = lax.axis_index("s")

  @pl.when(c == 0)
  def _():
    for k in range(LCH // 16):
      ones_v[pl.ds(k * 16, 16)] = jnp.ones((16,), jnp.int32)

    def zb(i, carry):
      zbuf[pl.ds(i * 16, 16)] = jnp.zeros((16,), jnp.int32)
      return carry
    lax.fori_loop(0, CRPT // 16, zb, 0, unroll=False)
    pltpu.sync_copy(zbuf, acc.at[pl.ds(s * CRPT, CRPT)])
    plsc.subcore_barrier()

    def chunk(j, carry):
      pltpu.sync_copy(dst2.at[s * 2 * CPT + j], dcur)
      pltpu.sync_copy(ones_v, acc.at[dcur], add=True)
      return carry
    lax.fori_loop(0, 2 * CPT, chunk, 0, unroll=False)
    plsc.subcore_barrier()

    @pl.when(s < NT - 1)
    def _():
      pltpu.sync_copy(acc.at[pl.ds(s * CRPT, CRPT)], zbuf)
      pltpu.sync_copy(zbuf, out.at[pl.ds(s * CRPT, CRPT)])
    @pl.when(s == NT - 1)
    def _():
      nlast = N - (NT - 1) * CRPT
      pltpu.sync_copy(acc.at[pl.ds((NT - 1) * CRPT, nlast)],
                      zbuf.at[pl.ds(0, nlast)])
      pltpu.sync_copy(zbuf.at[pl.ds(0, nlast)],
                      out.at[pl.ds((NT - 1) * CRPT, nlast)])

  @pl.when(c == 1)
  def _():
    plsc.subcore_barrier()
    plsc.subcore_barrier()


_count = pl.kernel(
    _count_body,
    out_type=jax.ShapeDtypeStruct((N,), jnp.int32),
    mesh=_MESH,
    scratch_types=[
        pltpu.VMEM((LCH,), jnp.int32),
        pltpu.VMEM((CRPT,), jnp.int32),
        pltpu.VMEM((LCH,), jnp.int32),
        pltpu.VMEM_SHARED((CNPAD,), jnp.int32),
    ],
    compiler_params=_SC_PARAMS,
)


BROWS = 1000
NBLK = N // BROWS


def _layer_body(relu, p0, p1, ha, hb, hc, cnt, wl, wr, b, oa, ob, oc):
  inv = 1.0 / jnp.maximum(cnt[...], 1.0)
  hp = jax.lax.Precision.HIGHEST
  dot = functools.partial(jnp.dot, precision=hp,
                          preferred_element_type=jnp.float32)
  wl_a = wl[...]
  wr_a = wr[...]
  p0_a = p0[...]
  p1_a = p1[...]
  y = jnp.broadcast_to(b[...], (BROWS, DP))
  for k, h_ref in enumerate((ha, hb, hc)):
    m = (p0_a[k] + p1_a[k]) * inv
    y = y + dot(m, wl_a[k * SW:(k + 1) * SW, :])
    y = y + dot(h_ref[...], wr_a[k * SW:(k + 1) * SW, :])
  if relu:
    y = jnp.maximum(y, 0.0)
  oa[...] = y[:, :SW]
  ob[...] = y[:, SW:2 * SW]
  oc[...] = y[:, 2 * SW:]


def _make_layer(relu):
  rs = lambda w: pl.BlockSpec((BROWS, w), lambda i: (i, 0))
  p_spec = pl.BlockSpec((NS, BROWS, SW), lambda i: (0, i, 0))
  w_spec = pl.BlockSpec((DP, DP), lambda i: (0, 0))
  b_spec = pl.BlockSpec((1, DP), lambda i: (0, 0))
  return pl.pallas_call(
      functools.partial(_layer_body, relu),
      grid=(NBLK,),
      in_specs=[p_spec, p_spec, rs(SW), rs(SW), rs(SW), rs(1),
                w_spec, w_spec, b_spec],
      out_specs=[rs(SW), rs(SW), rs(SW)],
      out_shape=[jax.ShapeDtypeStruct((N, SW), jnp.float32)] * 3,
      compiler_params=pltpu.CompilerParams(
          dimension_semantics=("arbitrary",)),
  )


_layer_relu = _make_layer(True)
_layer_lin = _make_layer(False)


def _pool_body(ha, hb, hc, bat, oa, ob, oc, gc):
  i = pl.program_id(0)

  @pl.when(i == 0)
  def _():
    oa[...] = jnp.zeros_like(oa)
    ob[...] = jnp.zeros_like(ob)
    oc[...] = jnp.zeros_like(oc)
    gc[...] = jnp.zeros_like(gc)

  ids = lax.broadcasted_iota(jnp.int32, (G, BROWS), 0)
  oneh = (ids == bat[0]).astype(jnp.float32)
  hp = jax.lax.Precision.HIGHEST
  dot = functools.partial(jnp.dot, precision=hp,
                          preferred_element_type=jnp.float32)
  oa[...] += dot(oneh, ha[...])
  ob[...] += dot(oneh, hb[...])
  oc[...] += dot(oneh, hc[...])
  gc[...] += jnp.sum(oneh, axis=1, keepdims=True)

  @pl.when(i == NBLK - 1)
  def _():
    g = jnp.maximum(gc[...], 1.0)
    oa[...] = oa[...] / g
    ob[...] = ob[...] / g
    oc[...] = oc[...] / g


_pool = pl.pallas_call(
    _pool_body,
    grid=(NBLK,),
    in_specs=[pl.BlockSpec((BROWS, SW), lambda i: (i, 0)),
              pl.BlockSpec((BROWS, SW), lambda i: (i, 0)),
              pl.BlockSpec((BROWS, SW), lambda i: (i, 0)),
              pl.BlockSpec((1, 1, BROWS), lambda i: (i, 0, 0))],
    out_specs=[pl.BlockSpec((G, SW), lambda i: (0, 0))] * 3,
    out_shape=[jax.ShapeDtypeStruct((G, SW), jnp.float32)] * 3,
    scratch_shapes=[pltpu.VMEM((G, 1), jnp.float32)],
    compiler_params=pltpu.CompilerParams(
        dimension_semantics=("arbitrary",)),
)


def kernel(x, edge_index, batch, W_l0, b_l0, W_r0, W_l1, b_l1, W_r1,
           W_l2, b_l2, W_r2):
  src = edge_index[0]
  dst = edge_index[1]
  pad = EPAD - E
  src2 = jnp.concatenate([src, jnp.zeros((pad,), jnp.int32)]
                         ).reshape(TOT_CHUNKS, LCH)
  dst2 = jnp.concatenate([dst, jnp.full((pad,), N, jnp.int32)]
                         ).reshape(TOT_CHUNKS, LCH)
  zrows = jnp.zeros((ZR, SW), jnp.float32)

  cnt = _count(dst2)
  cntf = cnt.astype(jnp.float32).reshape(N, 1)

  ha = x[:, :SW]
  hb = x[:, SW:2 * SW]
  hc = jnp.pad(x[:, 2 * SW:], ((0, 0), (0, DP - D)))

  weights = [(W_l0, b_l0, W_r0), (W_l1, b_l1, W_r1), (W_l2, b_l2, W_r2)]
  for l, (wl, bl, wr) in enumerate(weights):
    p0, p1 = _seg_sum(ha, hb, hc, src2, dst2, zrows)
    layer = _layer_relu if l < 2 else _layer_lin
    wlp = jnp.pad(wl, ((0, DP - D), (0, DP - D)))
    wrp = jnp.pad(wr, ((0, DP - D), (0, DP - D)))
    blp = jnp.pad(bl, (0, DP - D)).reshape(1, DP)
    ha, hb, hc = layer(p0, p1, ha, hb, hc, cntf, wlp, wrp, blp)

  batch3 = batch.reshape(NBLK, 1, BROWS)
  oa, ob, oc = _pool(ha, hb, hc, batch3)
  return jnp.concatenate([oa, ob, oc], axis=1)[:, :D]
